# baseline (speedup 1.0000x reference)
import functools

import jax
import jax.numpy as jnp
from jax import lax
from jax.experimental import pallas as pl
from jax.experimental.pallas import tpu as pltpu
from jax.experimental.pallas import tpu_sc as plsc

N = 10000
E = 320000
D_IN = 128
D_HID = 64
D_OUT = 128
F = 128

NC = 2
NS = 16
NW = NC * NS
EPW = E // NW
K = 80
CH = EPW // K

NA = 10240
RPT = NA // NS
R = 400
GRID = N // R


def _sc_body(p_hbm, src_hbm, dst_hbm, z128, out_ref, *rest):
    sets = tuple((rest[3 * k], rest[3 * k + 1], rest[3 * k + 2],
                  rest[9 + 2 * k], rest[10 + 2 * k]) for k in range(3))
    acc_sh = rest[15]
    c = lax.axis_index("c")
    s = lax.axis_index("s")
    wid = s * NC + c
    base = wid * EPW

    pltpu.sync_copy(z128, acc_sh.at[pl.ds(s * RPT, RPT)])
    plsc.subcore_barrier()

    def load_idx(i, st):
        b = base + i * K
        pltpu.sync_copy(src_hbm.at[pl.ds(b, K)], st[0])
        pltpu.sync_copy(dst_hbm.at[pl.ds(b, K)], st[1])

    def gather(i, st):
        return pltpu.make_async_copy(p_hbm.at[st[0]], st[2], st[3])

    def scat(st):
        return pltpu.make_async_copy(st[2], acc_sh.at[st[1]], st[4])

    def slot(i, p, wait_prev, issue_next, last=False):
        stP = sets[p]
        stR = sets[(p + 1) % 3]
        gather(i, stP).wait()
        pltpu.async_copy(stP[2], acc_sh.at[stP[1]], stP[4], add=True)
        if wait_prev:
            scat(stR).wait()
        if not last:
            load_idx(lax.min(i + 1, CH - 1), stR)
        if issue_next:
            gather(i + 1, stR).start()

    load_idx(0, sets[0])
    gather(0, sets[0]).start()
    slot(0, 0, False, True)
    slot(1, 1, False, True)

    def body(g, carry):
        i = 3 * g + 2
        slot(i, 2, True, True)
        slot(i + 1, 0, True, True)
        slot(i + 2, 1, True, True)
        return carry

    lax.fori_loop(0, (CH - 5) // 3, body, 0, unroll=False)
    slot(CH - 3, (CH - 3) % 3, True, True)
    slot(CH - 2, (CH - 2) % 3, True, True)
    slot(CH - 1, (CH - 1) % 3, True, False, last=True)
    scat(sets[(CH - 2) % 3]).wait()
    scat(sets[(CH - 1) % 3]).wait()

    plsc.subcore_barrier()

    pltpu.sync_copy(acc_sh.at[pl.ds(s * RPT, RPT)],
                    out_ref.at[c, pl.ds(s * RPT, RPT)])


def _make_sc_segsum():
    mesh = plsc.VectorSubcoreMesh(core_axis_name="c", subcore_axis_name="s",
                                  num_cores=NC, num_subcores=NS)
    out_type = jax.ShapeDtypeStruct((NC, NA, F), jnp.float32)
    scratch = (
        [pltpu.VMEM((K,), jnp.int32),
         pltpu.VMEM((K,), jnp.int32),
         pltpu.VMEM((K, F), jnp.float32)] * 3 +
        [pltpu.SemaphoreType.DMA] * 6 +
        [pltpu.VMEM_SHARED((NA, F), jnp.float32)]
    )
    return pl.kernel(_sc_body, out_type=out_type, mesh=mesh,
                     scratch_types=scratch)


def _full(shape):
    return pl.BlockSpec(shape, lambda i: tuple(0 for _ in shape))


def _rowspec(w):
    return pl.BlockSpec((R, w), lambda i: (i, 0))


def _accspec(core):
    return pl.BlockSpec((1, R, F), lambda i, _c=core: (_c, i, 0))


def _p1_body(x_ref, w1l, o_ref):
    p = jnp.dot(x_ref[...], w1l[...], preferred_element_type=jnp.float32)
    one = jnp.ones((R, 1), jnp.float32)
    zero = jnp.zeros((R, F - D_HID - 1), jnp.float32)
    o_ref[...] = jnp.concatenate([p, one, zero], axis=1)


def _p1(x, W1l):
    return pl.pallas_call(
        _p1_body,
        grid=(GRID,),
        in_specs=[_rowspec(D_IN), _full((D_IN, D_HID))],
        out_specs=_rowspec(F),
        out_shape=jax.ShapeDtypeStruct((N, F), jnp.float32),
    )(x, W1l)


def _inv_from(a0, a1):
    deg = a0[0, :, D_HID:D_HID + 1] + a1[0, :, D_HID:D_HID + 1]
    return 1.0 / jnp.maximum(deg, 1.0)


def _c1_body(a0, a1, x_ref, w1r, b1, w2l, h1_ref, p2_ref):
    inv = _inv_from(a0, a1)
    agg = (a0[0, :, :D_HID] + a1[0, :, :D_HID]) * inv
    h1 = jnp.maximum(
        agg + b1[...]
        + jnp.dot(x_ref[...], w1r[...], preferred_element_type=jnp.float32),
        0.0)
    h1_ref[...] = h1
    p2 = jnp.dot(h1, w2l[...], preferred_element_type=jnp.float32)
    p2_ref[...] = jnp.concatenate(
        [p2, jnp.zeros((R, F - D_HID), jnp.float32)], axis=1)


def _c1(acc1, x, W1r, b1, W2l):
    return pl.pallas_call(
        _c1_body,
        grid=(GRID,),
        in_specs=[_accspec(0), _accspec(1), _rowspec(D_IN),
                  _full((D_IN, D_HID)), _full((1, D_HID)),
                  _full((D_HID, D_HID))],
        out_specs=[_rowspec(D_HID), _rowspec(F)],
        out_shape=[jax.ShapeDtypeStruct((N, D_HID), jnp.float32),
                   jax.ShapeDtypeStruct((N, F), jnp.float32)],
    )(acc1, acc1, x, W1r, b1, W2l)


def _c2_body(a0, a1, d0, d1, h1_ref, w2r, b2, h2_ref):
    inv = _inv_from(d0, d1)
    agg = (a0[0, :, :D_HID] + a1[0, :, :D_HID]) * inv
    h2 = jnp.maximum(
        agg + b2[...]
        + jnp.dot(h1_ref[...], w2r[...], preferred_element_type=jnp.float32),
        0.0)
    h2_ref[...] = jnp.concatenate(
        [h2, jnp.zeros((R, F - D_HID), jnp.float32)], axis=1)


def _c2(acc2, acc1, h1, W2r, b2):
    return pl.pallas_call(
        _c2_body,
        grid=(GRID,),
        in_specs=[_accspec(0), _accspec(1), _accspec(0), _accspec(1),
                  _rowspec(D_HID), _full((D_HID, D_HID)), _full((1, D_HID))],
        out_specs=_rowspec(F),
        out_shape=jax.ShapeDtypeStruct((N, F), jnp.float32),
    )(acc2, acc2, acc1, acc1, h1, W2r, b2)


def _c3_body(a0, a1, d0, d1, h2_ref, w3l, b3, w3r, o_ref):
    inv = _inv_from(d0, d1)
    agg = (a0[0, :, :D_HID] + a1[0, :, :D_HID]) * inv
    o_ref[...] = (
        jnp.dot(agg, w3l[...], preferred_element_type=jnp.float32)
        + b3[...]
        + jnp.dot(h2_ref[...], w3r[...], preferred_element_type=jnp.float32))


def _c3(acc3, acc1, h2p, W3l, b3, W3rp):
    return pl.pallas_call(
        _c3_body,
        grid=(GRID,),
        in_specs=[_accspec(0), _accspec(1), _accspec(0), _accspec(1),
                  _rowspec(F), _full((D_HID, D_OUT)), _full((1, D_OUT)),
                  _full((F, D_OUT))],
        out_specs=_rowspec(D_OUT),
        out_shape=jax.ShapeDtypeStruct((N, D_OUT), jnp.float32),
    )(acc3, acc3, acc1, acc1, h2p, W3l, b3, W3rp)


def kernel(x, index, W1l, b1, W1r, W2l, b2, W2r, W3l, b3, W3r):
    src = index[0].astype(jnp.int32)
    dst = index[1].astype(jnp.int32)
    z128 = jnp.zeros((RPT, F), jnp.float32)
    W3rp = jnp.concatenate(
        [W3r, jnp.zeros((F - D_HID, D_OUT), jnp.float32)], axis=0)
    b1r = b1.reshape(1, D_HID)
    b2r = b2.reshape(1, D_HID)
    b3r = b3.reshape(1, D_OUT)

    segsum = _make_sc_segsum()

    p1 = _p1(x, W1l)
    acc1 = segsum(p1, src, dst, z128)
    h1, p2 = _c1(acc1, x, W1r, b1r, W2l)
    acc2 = segsum(p2, src, dst, z128)
    h2p = _c2(acc2, acc1, h1, W2r, b2r)
    acc3 = segsum(h2p, src, dst, z128)
    return _c3(acc3, acc1, h2p, W3l, b3r, W3rp)


# speedup vs baseline: 1.2814x; 1.2814x over previous
import functools

import jax
import jax.numpy as jnp
from jax import lax
from jax.experimental import pallas as pl
from jax.experimental.pallas import tpu as pltpu
from jax.experimental.pallas import tpu_sc as plsc

N = 10000
E = 320000
D_IN = 128
D_HID = 64
D_OUT = 128
F = 128

NC = 2
NS = 16
NW = NC * NS
EPW = E // NW
K = 80
CH = EPW // K

NA = 10240
RPT = NA // NS
R = 400
GRID = N // R


def _sc_body(p_hbm, src_hbm, dst_hbm, z128,
             out_ref, srcb, dstb, srcb1, dstb1, rows, acc_sh, gsem):
    c = lax.axis_index("c")
    s = lax.axis_index("s")
    wid = s * NC + c
    base = wid * EPW

    pltpu.sync_copy(z128, acc_sh.at[pl.ds(s * RPT, RPT)])
    plsc.subcore_barrier()

    def load_idx(i, sb, db):
        b = base + i * K
        pltpu.sync_copy(src_hbm.at[pl.ds(b, K)], sb)
        pltpu.sync_copy(dst_hbm.at[pl.ds(b, K)], db)

    def half(i, cur, nxt):
        sb, db = cur
        gat = pltpu.make_async_copy(p_hbm.at[sb], rows, gsem)
        gat.start()
        load_idx(lax.min(i + 1, CH - 1), *nxt)
        gat.wait()
        pltpu.sync_copy(rows, acc_sh.at[db], add=True)

    b0 = (srcb, dstb)
    b1 = (srcb1, dstb1)
    load_idx(0, *b0)

    def body(g, carry):
        half(2 * g, b0, b1)
        half(2 * g + 1, b1, b0)
        return carry

    lax.fori_loop(0, CH // 2, body, 0, unroll=False)
    half(CH - 1, b0, b1)
    plsc.subcore_barrier()

    pltpu.sync_copy(acc_sh.at[pl.ds(s * RPT, RPT)],
                    out_ref.at[c, pl.ds(s * RPT, RPT)])


def _make_sc_segsum():
    mesh = plsc.VectorSubcoreMesh(core_axis_name="c", subcore_axis_name="s",
                                  num_cores=NC, num_subcores=NS)
    out_type = jax.ShapeDtypeStruct((NC, NA, F), jnp.float32)
    scratch = [
        pltpu.VMEM((K,), jnp.int32),
        pltpu.VMEM((K,), jnp.int32),
        pltpu.VMEM((K,), jnp.int32),
        pltpu.VMEM((K,), jnp.int32),
        pltpu.VMEM((K, F), jnp.float32),
        pltpu.VMEM_SHARED((NA, F), jnp.float32),
        pltpu.SemaphoreType.DMA,
    ]
    return pl.kernel(_sc_body, out_type=out_type, mesh=mesh,
                     scratch_types=scratch)


def _p1_body(x_ref, w1l, o_ref):
    p = jnp.dot(x_ref[...], w1l[...], preferred_element_type=jnp.float32)
    one = jnp.ones((N, 1), jnp.float32)
    zero = jnp.zeros((N, F - D_HID - 1), jnp.float32)
    o_ref[...] = jnp.concatenate([p, one, zero], axis=1)


def _p1(x, W1l):
    return pl.pallas_call(
        _p1_body,
        out_shape=jax.ShapeDtypeStruct((N, F), jnp.float32),
    )(x, W1l)


def _agg_from(acc_ref, deg_ref):
    deg = (deg_ref[0, :N, D_HID:D_HID + 1]
           + deg_ref[1, :N, D_HID:D_HID + 1])
    inv = 1.0 / jnp.maximum(deg, 1.0)
    return (acc_ref[0, :N, :D_HID] + acc_ref[1, :N, :D_HID]) * inv


def _c1_body(acc1, x_ref, w1r, b1, w2l, h1_ref, p2_ref):
    agg = _agg_from(acc1, acc1)
    h1 = jnp.maximum(
        agg + b1[...]
        + jnp.dot(x_ref[...], w1r[...], preferred_element_type=jnp.float32),
        0.0)
    h1_ref[...] = h1
    p2 = jnp.dot(h1, w2l[...], preferred_element_type=jnp.float32)
    p2_ref[...] = jnp.concatenate(
        [p2, jnp.zeros((N, F - D_HID), jnp.float32)], axis=1)


def _c1(acc1, x, W1r, b1, W2l):
    return pl.pallas_call(
        _c1_body,
        out_shape=[jax.ShapeDtypeStruct((N, D_HID), jnp.float32),
                   jax.ShapeDtypeStruct((N, F), jnp.float32)],
    )(acc1, x, W1r, b1, W2l)


def _c2_body(acc2, acc1, h1_ref, w2r, b2, h2_ref):
    agg = _agg_from(acc2, acc1)
    h2 = jnp.maximum(
        agg + b2[...]
        + jnp.dot(h1_ref[...], w2r[...], preferred_element_type=jnp.float32),
        0.0)
    h2_ref[...] = jnp.concatenate(
        [h2, jnp.zeros((N, F - D_HID), jnp.float32)], axis=1)


def _c2(acc2, acc1, h1, W2r, b2):
    return pl.pallas_call(
        _c2_body,
        out_shape=jax.ShapeDtypeStruct((N, F), jnp.float32),
    )(acc2, acc1, h1, W2r, b2)


def _c3_body(acc3, acc1, h2_ref, w3l, b3, w3r, o_ref):
    agg = _agg_from(acc3, acc1)
    o_ref[...] = (
        jnp.dot(agg, w3l[...], preferred_element_type=jnp.float32)
        + b3[...]
        + jnp.dot(h2_ref[...], w3r[...], preferred_element_type=jnp.float32))


def _c3(acc3, acc1, h2p, W3l, b3, W3rp):
    return pl.pallas_call(
        _c3_body,
        out_shape=jax.ShapeDtypeStruct((N, D_OUT), jnp.float32),
    )(acc3, acc1, h2p, W3l, b3, W3rp)


def kernel(x, index, W1l, b1, W1r, W2l, b2, W2r, W3l, b3, W3r):
    src = index[0].astype(jnp.int32)
    dst = index[1].astype(jnp.int32)
    z128 = jnp.zeros((RPT, F), jnp.float32)
    W3rp = jnp.concatenate(
        [W3r, jnp.zeros((F - D_HID, D_OUT), jnp.float32)], axis=0)
    b1r = b1.reshape(1, D_HID)
    b2r = b2.reshape(1, D_HID)
    b3r = b3.reshape(1, D_OUT)

    segsum = _make_sc_segsum()

    p1 = _p1(x, W1l)
    acc1 = segsum(p1, src, dst, z128)
    h1, p2 = _c1(acc1, x, W1r, b1r, W2l)
    acc2 = segsum(p2, src, dst, z128)
    h2p = _c2(acc2, acc1, h1, W2r, b2r)
    acc3 = segsum(h2p, src, dst, z128)
    return _c3(acc3, acc1, h2p, W3l, b3r, W3rp)
